# baseline (speedup 1.0000x reference)
import functools

import jax
import jax.numpy as jnp
from jax.experimental import pallas as pl
from jax.experimental.pallas import tpu as pltpu

_BATCH = 1024
_D = 16
_BLOCK_N = 2048


def _proj_ln_kernel(user_t_ref, w_ref, b_ref, gamma_ref, beta_ref, h_ref):
    ht = jnp.dot(w_ref[:], user_t_ref[:],
                 preferred_element_type=jnp.float32) + b_ref[:]
    mu = jnp.mean(ht, axis=0, keepdims=True)
    d = ht - mu
    var = jnp.mean(d * d, axis=0, keepdims=True)
    ht = d * jax.lax.rsqrt(var + 1e-5) * gamma_ref[:] + beta_ref[:]
    h_ref[:] = ht.T


def _score_kernel(h_ref, item_t_ref, out_ref):
    out_ref[:] = jax.lax.dot_general(
        item_t_ref[:], h_ref[:], (((0,), (1,)), ((), ())),
        preferred_element_type=jnp.float32)


@jax.jit
def kernel(user_emb, item_emb, W, b, gamma, beta):
    n_items = item_emb.shape[0]
    user_t = user_emb.T
    item_t = item_emb.T
    h = pl.pallas_call(
        _proj_ln_kernel,
        out_shape=jax.ShapeDtypeStruct((_BATCH, _D), jnp.float32),
    )(user_t, W, b.reshape(_D, 1), gamma.reshape(_D, 1), beta.reshape(_D, 1))
    grid = (pl.cdiv(n_items, _BLOCK_N),)
    out_t = pl.pallas_call(
        _score_kernel,
        grid=grid,
        in_specs=[
            pl.BlockSpec((_BATCH, _D), lambda i: (0, 0)),
            pl.BlockSpec((_D, _BLOCK_N), lambda i: (0, i)),
        ],
        out_specs=pl.BlockSpec((_BLOCK_N, _BATCH), lambda i: (i, 0)),
        out_shape=jax.ShapeDtypeStruct((n_items, _BATCH), jnp.float32),
        compiler_params=pltpu.CompilerParams(
            dimension_semantics=("arbitrary",)),
    )(h, item_t)
    return out_t.T


# speedup vs baseline: 1.0341x; 1.0341x over previous
import functools

import jax
import jax.numpy as jnp
from jax.experimental import pallas as pl
from jax.experimental.pallas import tpu as pltpu

_BATCH = 1024
_D = 16
_BLOCK_N = 2048


def _fused_kernel(user_ref, w_ref, b_ref, gamma_ref, beta_ref, item_t_ref,
                  out_ref):
    h = jnp.dot(user_ref[:], w_ref[:].T,
                preferred_element_type=jnp.float32) + b_ref[:]
    mu = jnp.mean(h, axis=-1, keepdims=True)
    d = h - mu
    var = jnp.mean(d * d, axis=-1, keepdims=True)
    h = d * jax.lax.rsqrt(var + 1e-5) * gamma_ref[:] + beta_ref[:]
    out_ref[:] = jax.lax.dot_general(
        item_t_ref[:], h, (((0,), (1,)), ((), ())),
        preferred_element_type=jnp.float32)


@jax.jit
def kernel(user_emb, item_emb, W, b, gamma, beta):
    n_items = item_emb.shape[0]
    item_t = item_emb.T
    grid = (pl.cdiv(n_items, _BLOCK_N),)
    b2 = b.reshape(1, _D)
    gamma2 = gamma.reshape(1, _D)
    beta2 = beta.reshape(1, _D)
    out_t = pl.pallas_call(
        _fused_kernel,
        grid=grid,
        in_specs=[
            pl.BlockSpec((_BATCH, _D), lambda i: (0, 0)),
            pl.BlockSpec((_D, _D), lambda i: (0, 0)),
            pl.BlockSpec((1, _D), lambda i: (0, 0)),
            pl.BlockSpec((1, _D), lambda i: (0, 0)),
            pl.BlockSpec((1, _D), lambda i: (0, 0)),
            pl.BlockSpec((_D, _BLOCK_N), lambda i: (0, i)),
        ],
        out_specs=pl.BlockSpec((_BLOCK_N, _BATCH), lambda i: (i, 0)),
        out_shape=jax.ShapeDtypeStruct((n_items, _BATCH), jnp.float32),
        compiler_params=pltpu.CompilerParams(
            dimension_semantics=("arbitrary",)),
    )(user_emb, W, b2, gamma2, beta2, item_t)
    return out_t.T


# speedup vs baseline: 1.0478x; 1.0132x over previous
import jax
import jax.numpy as jnp
from jax.experimental import pallas as pl
from jax.experimental.pallas import tpu as pltpu

_BATCH = 1024
_D = 16
_BLOCK_N = 2048


def _fused_kernel(user_ref, w_ref, b_ref, gamma_ref, beta_ref, item_t_ref,
                  out_ref):
    h = jnp.dot(user_ref[:], w_ref[:].T,
                preferred_element_type=jnp.float32) + b_ref[:]
    mu = jnp.mean(h, axis=-1, keepdims=True)
    d = h - mu
    var = jnp.mean(d * d, axis=-1, keepdims=True)
    h = d * jax.lax.rsqrt(var + 1e-5) * gamma_ref[:] + beta_ref[:]
    out_ref[:] = jax.lax.dot_general(
        item_t_ref[:], h, (((0,), (1,)), ((), ())),
        preferred_element_type=jnp.float32)


@jax.jit
def kernel(user_emb, item_emb, W, b, gamma, beta):
    n_items = item_emb.shape[0]
    item_t = item_emb.T
    grid = (pl.cdiv(n_items, _BLOCK_N),)
    b2 = b.reshape(1, _D)
    gamma2 = gamma.reshape(1, _D)
    beta2 = beta.reshape(1, _D)
    out_t = pl.pallas_call(
        _fused_kernel,
        grid=grid,
        in_specs=[
            pl.BlockSpec((_BATCH, _D), lambda i: (0, 0)),
            pl.BlockSpec((_D, _D), lambda i: (0, 0)),
            pl.BlockSpec((1, _D), lambda i: (0, 0)),
            pl.BlockSpec((1, _D), lambda i: (0, 0)),
            pl.BlockSpec((1, _D), lambda i: (0, 0)),
            pl.BlockSpec((_D, _BLOCK_N), lambda i: (0, i)),
        ],
        out_specs=pl.BlockSpec((_BLOCK_N, _BATCH), lambda i: (i, 0)),
        out_shape=jax.ShapeDtypeStruct((n_items, _BATCH), jnp.float32),
        compiler_params=pltpu.CompilerParams(
            dimension_semantics=("arbitrary",)),
    )(user_emb, W, b2, gamma2, beta2, item_t)
    return out_t.T
